# Initial kernel scaffold; baseline (speedup 1.0000x reference)
import jax
import jax.numpy as jnp
from jax.experimental import pallas as pl


def kernel(topic_one_hot, edge_index, reverse_edge_index):
    raise NotImplementedError("write your pallas kernel here")


# speedup vs baseline: 116.4549x; 116.4549x over previous
import jax
import jax.numpy as jnp
from jax import lax
from jax.experimental import pallas as pl
from jax.experimental.pallas import tpu as pltpu
from jax.experimental.pallas import tpu_sc as plsc

N_NODES = 100000
N_EDGES = 6400000
N_ROUNDS = 8
NTILES = 16
NPAD = 102400
NC = NPAD // NTILES
E_TILE = N_EDGES // NTILES
B = 8000
NCHUNK = E_TILE // B


def _dde_body(x0pad, x1pad, srcs, dsts, ones, zeros, out0, out1,
              x0_sp, x1_sp, a0_sp, a1_sp, cnt_sp,
              src_buf, dst_buf, v0_buf, v1_buf, b0_buf, b1_buf,
              inv_buf, ones_buf):
    cid = lax.axis_index("c")
    tid = lax.axis_index("s")
    n0 = tid * NC
    e_base = cid * N_EDGES + tid * E_TILE

    pltpu.sync_copy(x0pad.at[pl.ds(n0, NC)], b0_buf)
    pltpu.sync_copy(b0_buf, x0_sp.at[pl.ds(n0, NC)])
    pltpu.sync_copy(x1pad.at[pl.ds(n0, NC)], b1_buf)
    pltpu.sync_copy(b1_buf, x1_sp.at[pl.ds(n0, NC)])
    pltpu.sync_copy(zeros.at[pl.ds(n0, NC)], inv_buf)
    pltpu.sync_copy(inv_buf, cnt_sp.at[pl.ds(n0, NC)])
    pltpu.sync_copy(ones, ones_buf)
    plsc.subcore_barrier()

    def round_body(r, carry):
        pltpu.sync_copy(zeros.at[pl.ds(n0, NC)], b0_buf)
        pltpu.sync_copy(b0_buf, a0_sp.at[pl.ds(n0, NC)])
        pltpu.sync_copy(b0_buf, a1_sp.at[pl.ds(n0, NC)])
        plsc.subcore_barrier()

        def chunk_body(i, c):
            e0 = e_base + i * B
            pltpu.sync_copy(srcs.at[pl.ds(e0, B)], src_buf)
            pltpu.sync_copy(dsts.at[pl.ds(e0, B)], dst_buf)
            pltpu.sync_copy(x0_sp.at[src_buf], v0_buf)
            pltpu.sync_copy(x1_sp.at[src_buf], v1_buf)
            pltpu.sync_copy(v0_buf, a0_sp.at[dst_buf], add=True)
            pltpu.sync_copy(v1_buf, a1_sp.at[dst_buf], add=True)

            @pl.when(r == 0)
            def _():
                pltpu.sync_copy(ones_buf, cnt_sp.at[dst_buf], add=True)

            return c

        lax.fori_loop(0, NCHUNK, chunk_body, 0)
        plsc.subcore_barrier()

        @pl.when(r == 0)
        def _():
            pltpu.sync_copy(cnt_sp.at[pl.ds(n0, NC)], inv_buf)

            def inv_body(j, c):
                v = inv_buf[pl.ds(j * 16, 16)]
                inv_buf[pl.ds(j * 16, 16)] = 1.0 / jnp.maximum(v, 1.0)
                return c

            lax.fori_loop(0, NC // 16, inv_body, 0)

        pltpu.sync_copy(a0_sp.at[pl.ds(n0, NC)], b0_buf)
        pltpu.sync_copy(a1_sp.at[pl.ds(n0, NC)], b1_buf)

        def norm_body(j, c):
            s = pl.ds(j * 16, 16)
            v = inv_buf[s]
            b0_buf[s] = b0_buf[s] * v
            b1_buf[s] = b1_buf[s] * v
            return c

        lax.fori_loop(0, NC // 16, norm_body, 0)

        pltpu.sync_copy(b0_buf, x0_sp.at[pl.ds(n0, NC)])
        pltpu.sync_copy(b1_buf, x1_sp.at[pl.ds(n0, NC)])
        pltpu.sync_copy(b0_buf, out0.at[cid, r, pl.ds(n0, NC)])
        pltpu.sync_copy(b1_buf, out1.at[cid, r, pl.ds(n0, NC)])
        plsc.subcore_barrier()
        return carry

    lax.fori_loop(0, N_ROUNDS, round_body, 0)


_mesh = plsc.VectorSubcoreMesh(core_axis_name="c", subcore_axis_name="s")

_dde_call = pl.kernel(
    _dde_body,
    out_type=(
        jax.ShapeDtypeStruct((2, N_ROUNDS, NPAD), jnp.float32),
        jax.ShapeDtypeStruct((2, N_ROUNDS, NPAD), jnp.float32),
    ),
    mesh=_mesh,
    scratch_types=[
        pltpu.VMEM_SHARED((NPAD,), jnp.float32),
        pltpu.VMEM_SHARED((NPAD,), jnp.float32),
        pltpu.VMEM_SHARED((NPAD,), jnp.float32),
        pltpu.VMEM_SHARED((NPAD,), jnp.float32),
        pltpu.VMEM_SHARED((NPAD,), jnp.float32),
        pltpu.VMEM((B,), jnp.int32),
        pltpu.VMEM((B,), jnp.int32),
        pltpu.VMEM((B,), jnp.float32),
        pltpu.VMEM((B,), jnp.float32),
        pltpu.VMEM((NC,), jnp.float32),
        pltpu.VMEM((NC,), jnp.float32),
        pltpu.VMEM((NC,), jnp.float32),
        pltpu.VMEM((B,), jnp.float32),
    ],
)


def kernel(topic_one_hot, edge_index, reverse_edge_index):
    x0pad = jnp.zeros((NPAD,), jnp.float32).at[:N_NODES].set(topic_one_hot[:, 0])
    x1pad = jnp.zeros((NPAD,), jnp.float32).at[:N_NODES].set(topic_one_hot[:, 1])
    srcs = jnp.concatenate([edge_index[0], reverse_edge_index[0]])
    dsts = jnp.concatenate([edge_index[1], reverse_edge_index[1]])
    ones = jnp.ones((B,), jnp.float32)
    zeros = jnp.zeros((NPAD,), jnp.float32)
    out0, out1 = _dde_call(x0pad, x1pad, srcs, dsts, ones, zeros)
    res = []
    for c in range(2):
        for r in range(N_ROUNDS):
            res.append(jnp.stack([out0[c, r, :N_NODES], out1[c, r, :N_NODES]],
                                 axis=-1))
    return tuple(res)


# speedup vs baseline: 117.2901x; 1.0072x over previous
import jax
import jax.numpy as jnp
from jax import lax
from jax.experimental import pallas as pl
from jax.experimental.pallas import tpu as pltpu
from jax.experimental.pallas import tpu_sc as plsc

N_NODES = 100000
N_EDGES = 6400000
N_ROUNDS = 8
NTILES = 16
NPAD = 102400
NC = NPAD // NTILES
E_TILE = N_EDGES // NTILES
B = 8000
NCHUNK = E_TILE // B


def _dde_body(x0pad, x1pad, srcs, dsts, ones, zeros, out0, out1,
              x0_sp, x1_sp, a0_sp, a1_sp, cnt_sp,
              src_a, src_c, dst_a, dst_c, v0_a, v0_c, v1_a, v1_c,
              b0_buf, b1_buf, inv_buf, ones_buf,
              i_sem_a, i_sem_c, g_sem_a, g_sem_c,
              s_sem_a, s_sem_c, c_sem_a, c_sem_c):
    cid = lax.axis_index("c")
    tid = lax.axis_index("s")
    n0 = tid * NC
    e_base = cid * N_EDGES + tid * E_TILE

    src_b = [src_a, src_c]
    dst_b = [dst_a, dst_c]
    v0_b = [v0_a, v0_c]
    v1_b = [v1_a, v1_c]
    i_sem = [i_sem_a, i_sem_c]
    g_sem = [g_sem_a, g_sem_c]
    s_sem = [s_sem_a, s_sem_c]
    c_sem = [c_sem_a, c_sem_c]

    def issue_idx(i, s):
        ic = jnp.minimum(i, NCHUNK - 1)
        e0 = e_base + ic * B
        pltpu.async_copy(srcs.at[pl.ds(e0, B)], src_b[s], i_sem[s])
        pltpu.async_copy(dsts.at[pl.ds(e0, B)], dst_b[s], i_sem[s])

    def wait_idx(s):
        pltpu.make_async_copy(srcs.at[pl.ds(0, B)], src_b[s], i_sem[s]).wait()
        pltpu.make_async_copy(dsts.at[pl.ds(0, B)], dst_b[s], i_sem[s]).wait()

    def wait_scat(s):
        pltpu.make_async_copy(v0_b[s], a0_sp.at[dst_b[s]], s_sem[s]).wait()
        pltpu.make_async_copy(v1_b[s], a1_sp.at[dst_b[s]], s_sem[s]).wait()

    pltpu.sync_copy(x0pad.at[pl.ds(n0, NC)], b0_buf)
    pltpu.sync_copy(b0_buf, x0_sp.at[pl.ds(n0, NC)])
    pltpu.sync_copy(x1pad.at[pl.ds(n0, NC)], b1_buf)
    pltpu.sync_copy(b1_buf, x1_sp.at[pl.ds(n0, NC)])
    pltpu.sync_copy(zeros.at[pl.ds(n0, NC)], inv_buf)
    pltpu.sync_copy(inv_buf, cnt_sp.at[pl.ds(n0, NC)])
    pltpu.sync_copy(ones, ones_buf)
    plsc.subcore_barrier()

    def round_body(r, carry):
        issue_idx(0, 0)
        pltpu.sync_copy(zeros.at[pl.ds(n0, NC)], b0_buf)
        pltpu.sync_copy(b0_buf, a0_sp.at[pl.ds(n0, NC)])
        pltpu.sync_copy(b0_buf, a1_sp.at[pl.ds(n0, NC)])
        plsc.subcore_barrier()

        def chunk_pair(g, c):
            for b in (0, 1):
                i = 2 * g + b
                o = 1 - b
                wait_idx(b)
                pltpu.async_copy(x0_sp.at[src_b[b]], v0_b[b], g_sem[b])
                pltpu.async_copy(x1_sp.at[src_b[b]], v1_b[b], g_sem[b])

                @pl.when(i >= 1)
                def _():
                    wait_scat(o)

                @pl.when(jnp.logical_and(r == 0, i >= 1))
                def _():
                    pltpu.make_async_copy(
                        ones_buf, cnt_sp.at[dst_b[o]], c_sem[o]).wait()

                issue_idx(i + 1, o)
                pltpu.make_async_copy(x0_sp.at[src_b[b]], v0_b[b],
                                      g_sem[b]).wait()
                pltpu.make_async_copy(x1_sp.at[src_b[b]], v1_b[b],
                                      g_sem[b]).wait()
                pltpu.async_copy(v0_b[b], a0_sp.at[dst_b[b]], s_sem[b],
                                 add=True)
                pltpu.async_copy(v1_b[b], a1_sp.at[dst_b[b]], s_sem[b],
                                 add=True)

                @pl.when(r == 0)
                def _():
                    pltpu.async_copy(ones_buf, cnt_sp.at[dst_b[b]], c_sem[b],
                                     add=True)

            return c

        lax.fori_loop(0, NCHUNK // 2, chunk_pair, 0)
        wait_scat(1)
        wait_idx(0)

        @pl.when(r == 0)
        def _():
            pltpu.make_async_copy(ones_buf, cnt_sp.at[dst_b[1]],
                                  c_sem[1]).wait()

        plsc.subcore_barrier()

        @pl.when(r == 0)
        def _():
            pltpu.sync_copy(cnt_sp.at[pl.ds(n0, NC)], inv_buf)

            def inv_body(j, c):
                v = inv_buf[pl.ds(j * 16, 16)]
                inv_buf[pl.ds(j * 16, 16)] = 1.0 / jnp.maximum(v, 1.0)
                return c

            lax.fori_loop(0, NC // 16, inv_body, 0)

        pltpu.sync_copy(a0_sp.at[pl.ds(n0, NC)], b0_buf)
        pltpu.sync_copy(a1_sp.at[pl.ds(n0, NC)], b1_buf)

        def norm_body(j, c):
            s = pl.ds(j * 16, 16)
            v = inv_buf[s]
            b0_buf[s] = b0_buf[s] * v
            b1_buf[s] = b1_buf[s] * v
            return c

        lax.fori_loop(0, NC // 16, norm_body, 0)

        pltpu.sync_copy(b0_buf, x0_sp.at[pl.ds(n0, NC)])
        pltpu.sync_copy(b1_buf, x1_sp.at[pl.ds(n0, NC)])
        pltpu.sync_copy(b0_buf, out0.at[cid, r, pl.ds(n0, NC)])
        pltpu.sync_copy(b1_buf, out1.at[cid, r, pl.ds(n0, NC)])
        plsc.subcore_barrier()
        return carry

    lax.fori_loop(0, N_ROUNDS, round_body, 0)


_mesh = plsc.VectorSubcoreMesh(core_axis_name="c", subcore_axis_name="s")

_dde_call = pl.kernel(
    _dde_body,
    out_type=(
        jax.ShapeDtypeStruct((2, N_ROUNDS, NPAD), jnp.float32),
        jax.ShapeDtypeStruct((2, N_ROUNDS, NPAD), jnp.float32),
    ),
    mesh=_mesh,
    scratch_types=[
        pltpu.VMEM_SHARED((NPAD,), jnp.float32),
        pltpu.VMEM_SHARED((NPAD,), jnp.float32),
        pltpu.VMEM_SHARED((NPAD,), jnp.float32),
        pltpu.VMEM_SHARED((NPAD,), jnp.float32),
        pltpu.VMEM_SHARED((NPAD,), jnp.float32),
        pltpu.VMEM((B,), jnp.int32),
        pltpu.VMEM((B,), jnp.int32),
        pltpu.VMEM((B,), jnp.int32),
        pltpu.VMEM((B,), jnp.int32),
        pltpu.VMEM((B,), jnp.float32),
        pltpu.VMEM((B,), jnp.float32),
        pltpu.VMEM((B,), jnp.float32),
        pltpu.VMEM((B,), jnp.float32),
        pltpu.VMEM((NC,), jnp.float32),
        pltpu.VMEM((NC,), jnp.float32),
        pltpu.VMEM((NC,), jnp.float32),
        pltpu.VMEM((B,), jnp.float32),
        pltpu.SemaphoreType.DMA,
        pltpu.SemaphoreType.DMA,
        pltpu.SemaphoreType.DMA,
        pltpu.SemaphoreType.DMA,
        pltpu.SemaphoreType.DMA,
        pltpu.SemaphoreType.DMA,
        pltpu.SemaphoreType.DMA,
        pltpu.SemaphoreType.DMA,
    ],
)


def kernel(topic_one_hot, edge_index, reverse_edge_index):
    x0pad = jnp.zeros((NPAD,), jnp.float32).at[:N_NODES].set(topic_one_hot[:, 0])
    x1pad = jnp.zeros((NPAD,), jnp.float32).at[:N_NODES].set(topic_one_hot[:, 1])
    srcs = jnp.concatenate([edge_index[0], reverse_edge_index[0]])
    dsts = jnp.concatenate([edge_index[1], reverse_edge_index[1]])
    ones = jnp.ones((B,), jnp.float32)
    zeros = jnp.zeros((NPAD,), jnp.float32)
    out0, out1 = _dde_call(x0pad, x1pad, srcs, dsts, ones, zeros)
    res = []
    for c in range(2):
        for r in range(N_ROUNDS):
            res.append(jnp.stack([out0[c, r, :N_NODES], out1[c, r, :N_NODES]],
                                 axis=-1))
    return tuple(res)


# speedup vs baseline: 123.6765x; 1.0544x over previous
import jax
import jax.numpy as jnp
from jax import lax
from jax.experimental import pallas as pl
from jax.experimental.pallas import tpu as pltpu
from jax.experimental.pallas import tpu_sc as plsc

N_NODES = 100000
N_EDGES = 6400000
N_ROUNDS = 8
NTILES = 16
NPAD = 102400
NC = NPAD // NTILES
E_TILE = N_EDGES // NTILES
B = 8000
NCHUNK = E_TILE // B


def _dde_body(xp_hbm, srcs, dsts, ones, zeros, out0, out1,
              xp_sp, a0_sp, a1_sp, cnt_sp,
              src_a, src_c, dst_a, dst_c, vp_a, vp_c,
              v0_a, v0_c, v1_a, v1_c, ones_buf,
              i_sem_a, i_sem_c, g_sem_a, g_sem_c,
              s_sem_a, s_sem_c, c_sem_a, c_sem_c):
    cid = lax.axis_index("c")
    tid = lax.axis_index("s")
    n0 = tid * NC
    e_base = cid * N_EDGES + tid * E_TILE

    src_b = [src_a, src_c]
    dst_b = [dst_a, dst_c]
    vp_b = [vp_a, vp_c]
    v0_b = [v0_a, v0_c]
    v1_b = [v1_a, v1_c]
    i_sem = [i_sem_a, i_sem_c]
    g_sem = [g_sem_a, g_sem_c]
    s_sem = [s_sem_a, s_sem_c]
    c_sem = [c_sem_a, c_sem_c]

    def issue_idx(i, s):
        ic = jnp.minimum(i, NCHUNK - 1)
        e0 = e_base + ic * B
        pltpu.async_copy(srcs.at[pl.ds(e0, B)], src_b[s], i_sem[s])
        pltpu.async_copy(dsts.at[pl.ds(e0, B)], dst_b[s], i_sem[s])

    def wait_idx(s):
        pltpu.make_async_copy(srcs.at[pl.ds(0, B)], src_b[s], i_sem[s]).wait()
        pltpu.make_async_copy(dsts.at[pl.ds(0, B)], dst_b[s], i_sem[s]).wait()

    def wait_scat(s):
        pltpu.make_async_copy(v0_b[s], a0_sp.at[dst_b[s]], s_sem[s]).wait()
        pltpu.make_async_copy(v1_b[s], a1_sp.at[dst_b[s]], s_sem[s]).wait()

    pltpu.sync_copy(xp_hbm.at[pl.ds(n0, NC)], vp_a.at[pl.ds(0, NC)])
    pltpu.sync_copy(vp_a.at[pl.ds(0, NC)], xp_sp.at[pl.ds(n0, NC)])
    pltpu.sync_copy(zeros.at[pl.ds(n0, NC)], v0_a.at[pl.ds(0, NC)])
    pltpu.sync_copy(v0_a.at[pl.ds(0, NC)], cnt_sp.at[pl.ds(n0, NC)])
    pltpu.sync_copy(ones, ones_buf)
    plsc.subcore_barrier()

    def round_body(r, carry):
        pltpu.sync_copy(zeros.at[pl.ds(n0, NC)], v0_a.at[pl.ds(0, NC)])
        pltpu.sync_copy(v0_a.at[pl.ds(0, NC)], a0_sp.at[pl.ds(n0, NC)])
        pltpu.sync_copy(v0_a.at[pl.ds(0, NC)], a1_sp.at[pl.ds(n0, NC)])
        plsc.subcore_barrier()
        issue_idx(0, 0)

        def chunk_pair(g, c):
            for b in (0, 1):
                i = 2 * g + b
                o = 1 - b
                wait_idx(b)
                pltpu.async_copy(xp_sp.at[src_b[b]], vp_b[b], g_sem[b])
                pltpu.make_async_copy(xp_sp.at[src_b[b]], vp_b[b],
                                      g_sem[b]).wait()

                def unp_body(k, c2):
                    s4 = pl.ds(k * 16, 16)
                    wb = plsc.bitcast(vp_b[b][s4], jnp.bfloat16)
                    g0, g1 = plsc.unpack(
                        wb, format=plsc.PackFormat.INTERLEAVED)
                    v0_b[b][s4] = g0
                    v1_b[b][s4] = g1
                    return c2

                lax.fori_loop(0, B // 16, unp_body, 0)

                @pl.when(i >= 1)
                def _():
                    wait_scat(o)

                @pl.when(jnp.logical_and(r == 0, i >= 1))
                def _():
                    pltpu.make_async_copy(
                        ones_buf, cnt_sp.at[dst_b[o]], c_sem[o]).wait()

                issue_idx(i + 1, o)
                pltpu.async_copy(v0_b[b], a0_sp.at[dst_b[b]], s_sem[b],
                                 add=True)
                pltpu.async_copy(v1_b[b], a1_sp.at[dst_b[b]], s_sem[b],
                                 add=True)

                @pl.when(r == 0)
                def _():
                    pltpu.async_copy(ones_buf, cnt_sp.at[dst_b[b]], c_sem[b],
                                     add=True)

            return c

        lax.fori_loop(0, NCHUNK // 2, chunk_pair, 0)
        wait_scat(1)
        wait_idx(0)

        @pl.when(r == 0)
        def _():
            pltpu.make_async_copy(ones_buf, cnt_sp.at[dst_b[1]],
                                  c_sem[1]).wait()

        plsc.subcore_barrier()

        @pl.when(r == 0)
        def _():
            pltpu.sync_copy(cnt_sp.at[pl.ds(n0, NC)], v0_c.at[pl.ds(0, NC)])

            def inv_body(j, c2):
                s4 = pl.ds(j * 16, 16)
                v0_c[s4] = 1.0 / jnp.maximum(v0_c[s4], 1.0)
                return c2

            lax.fori_loop(0, NC // 16, inv_body, 0)
            pltpu.sync_copy(v0_c.at[pl.ds(0, NC)], cnt_sp.at[pl.ds(n0, NC)])

        pltpu.sync_copy(a0_sp.at[pl.ds(n0, NC)], v0_a.at[pl.ds(0, NC)])
        pltpu.sync_copy(a1_sp.at[pl.ds(n0, NC)], v1_a.at[pl.ds(0, NC)])
        pltpu.sync_copy(cnt_sp.at[pl.ds(n0, NC)], v0_c.at[pl.ds(0, NC)])

        def norm_body(j, c2):
            s4 = pl.ds(j * 16, 16)
            iv = v0_c[s4]
            o0 = v0_a[s4] * iv
            o1 = v1_a[s4] * iv
            v0_a[s4] = o0
            v1_a[s4] = o1
            pk = plsc.pack(o0, o1, format=plsc.PackFormat.INTERLEAVED)
            vp_a[s4] = plsc.bitcast(pk, jnp.float32)
            return c2

        lax.fori_loop(0, NC // 16, norm_body, 0)
        ooff = (cid * N_ROUNDS + r) * NPAD + n0
        pltpu.sync_copy(v0_a.at[pl.ds(0, NC)], out0.at[pl.ds(ooff, NC)])
        pltpu.sync_copy(v1_a.at[pl.ds(0, NC)], out1.at[pl.ds(ooff, NC)])
        pltpu.sync_copy(vp_a.at[pl.ds(0, NC)], xp_sp.at[pl.ds(n0, NC)])
        plsc.subcore_barrier()
        return carry

    lax.fori_loop(0, N_ROUNDS, round_body, 0)


_mesh = plsc.VectorSubcoreMesh(core_axis_name="c", subcore_axis_name="s")

_dde_call = pl.kernel(
    _dde_body,
    out_type=(
        jax.ShapeDtypeStruct((2 * N_ROUNDS * NPAD,), jnp.float32),
        jax.ShapeDtypeStruct((2 * N_ROUNDS * NPAD,), jnp.float32),
    ),
    mesh=_mesh,
    compiler_params=pltpu.CompilerParams(needs_layout_passes=False),
    scratch_types=[
        pltpu.VMEM_SHARED((NPAD,), jnp.float32),
        pltpu.VMEM_SHARED((NPAD,), jnp.float32),
        pltpu.VMEM_SHARED((NPAD,), jnp.float32),
        pltpu.VMEM_SHARED((NPAD,), jnp.float32),
        pltpu.VMEM((B,), jnp.int32),
        pltpu.VMEM((B,), jnp.int32),
        pltpu.VMEM((B,), jnp.int32),
        pltpu.VMEM((B,), jnp.int32),
        pltpu.VMEM((B,), jnp.float32),
        pltpu.VMEM((B,), jnp.float32),
        pltpu.VMEM((B,), jnp.float32),
        pltpu.VMEM((B,), jnp.float32),
        pltpu.VMEM((B,), jnp.float32),
        pltpu.VMEM((B,), jnp.float32),
        pltpu.VMEM((B,), jnp.float32),
        pltpu.SemaphoreType.DMA,
        pltpu.SemaphoreType.DMA,
        pltpu.SemaphoreType.DMA,
        pltpu.SemaphoreType.DMA,
        pltpu.SemaphoreType.DMA,
        pltpu.SemaphoreType.DMA,
        pltpu.SemaphoreType.DMA,
        pltpu.SemaphoreType.DMA,
    ],
)


def kernel(topic_one_hot, edge_index, reverse_edge_index):
    bits0 = lax.bitcast_convert_type(
        topic_one_hot[:, 0].astype(jnp.bfloat16), jnp.uint16
    ).astype(jnp.uint32)
    bits1 = lax.bitcast_convert_type(
        topic_one_hot[:, 1].astype(jnp.bfloat16), jnp.uint16
    ).astype(jnp.uint32)
    packed = lax.bitcast_convert_type(
        jnp.left_shift(bits1, 16) | bits0, jnp.float32)
    xp = jnp.zeros((NPAD,), jnp.float32).at[:N_NODES].set(packed)
    srcs = jnp.concatenate([edge_index[0], reverse_edge_index[0]])
    dsts = jnp.concatenate([edge_index[1], reverse_edge_index[1]])
    ones = jnp.ones((B,), jnp.float32)
    zeros = jnp.zeros((NPAD,), jnp.float32)
    out0, out1 = _dde_call(xp, srcs, dsts, ones, zeros)
    out0 = out0.reshape(2, N_ROUNDS, NPAD)
    out1 = out1.reshape(2, N_ROUNDS, NPAD)
    res = []
    for c in range(2):
        for r in range(N_ROUNDS):
            res.append(jnp.stack([out0[c, r, :N_NODES], out1[c, r, :N_NODES]],
                                 axis=-1))
    return tuple(res)


# speedup vs baseline: 133.9976x; 1.0835x over previous
import jax
import jax.numpy as jnp
from jax import lax
from jax.experimental import pallas as pl
from jax.experimental.pallas import tpu as pltpu
from jax.experimental.pallas import tpu_sc as plsc

N_NODES = 100000
N_EDGES = 6400000
N_ROUNDS = 8
NTILES = 16
NPAD = 102400
NC = NPAD // NTILES
E_TILE = N_EDGES // NTILES
B = 8000
NCHUNK = E_TILE // B


def _dde_body(xp_hbm, srcs, dsts, ones, zeros, out0, out1,
              xp_sp, a0_sp, a1_sp, cnt_sp,
              src_a, src_c, dst_a, dst_c, vp_a, vp_c,
              v0_a, v0_c, v1_a, v1_c, ones_buf,
              i_sem_a, i_sem_c, g_sem_a, g_sem_c,
              s_sem_a, s_sem_c, c_sem_a, c_sem_c):
    cid = lax.axis_index("c")
    tid = lax.axis_index("s")
    n0 = tid * NC
    e_base = cid * N_EDGES + tid * E_TILE

    src_b = [src_a, src_c]
    dst_b = [dst_a, dst_c]
    vp_b = [vp_a, vp_c]
    v0_b = [v0_a, v0_c]
    v1_b = [v1_a, v1_c]
    i_sem = [i_sem_a, i_sem_c]
    g_sem = [g_sem_a, g_sem_c]
    s_sem = [s_sem_a, s_sem_c]
    c_sem = [c_sem_a, c_sem_c]

    def issue_idx(i, s):
        ic = jnp.minimum(i, NCHUNK - 1)
        e0 = e_base + ic * B
        pltpu.async_copy(srcs.at[pl.ds(e0, B)], src_b[s], i_sem[s])
        pltpu.async_copy(dsts.at[pl.ds(e0, B)], dst_b[s], i_sem[s])

    def wait_idx(s):
        pltpu.make_async_copy(srcs.at[pl.ds(0, B)], src_b[s], i_sem[s]).wait()
        pltpu.make_async_copy(dsts.at[pl.ds(0, B)], dst_b[s], i_sem[s]).wait()

    def wait_scat(s):
        pltpu.make_async_copy(v0_b[s], a0_sp.at[dst_b[s]], s_sem[s]).wait()
        pltpu.make_async_copy(v1_b[s], a1_sp.at[dst_b[s]], s_sem[s]).wait()

    pltpu.sync_copy(xp_hbm.at[pl.ds(n0, NC)], vp_a.at[pl.ds(0, NC)])
    pltpu.sync_copy(vp_a.at[pl.ds(0, NC)], xp_sp.at[pl.ds(n0, NC)])
    pltpu.sync_copy(zeros.at[pl.ds(n0, NC)], v0_a.at[pl.ds(0, NC)])
    pltpu.sync_copy(v0_a.at[pl.ds(0, NC)], cnt_sp.at[pl.ds(n0, NC)])
    pltpu.sync_copy(ones, ones_buf)
    plsc.subcore_barrier()

    def round_body(r, carry):
        pltpu.sync_copy(zeros.at[pl.ds(n0, NC)], v0_a.at[pl.ds(0, NC)])
        pltpu.sync_copy(v0_a.at[pl.ds(0, NC)], a0_sp.at[pl.ds(n0, NC)])
        pltpu.sync_copy(v0_a.at[pl.ds(0, NC)], a1_sp.at[pl.ds(n0, NC)])
        plsc.subcore_barrier()
        issue_idx(0, 0)

        def chunk_pair(g, c):
            for b in (0, 1):
                i = 2 * g + b
                o = 1 - b
                wait_idx(b)
                pltpu.async_copy(xp_sp.at[src_b[b]], vp_b[b], g_sem[b])
                pltpu.make_async_copy(xp_sp.at[src_b[b]], vp_b[b],
                                      g_sem[b]).wait()

                def unp_body(k, c2):
                    for u in range(4):
                        s4 = pl.ds(k * 64 + u * 16, 16)
                        wb = plsc.bitcast(vp_b[b][s4], jnp.bfloat16)
                        g0, g1 = plsc.unpack(
                            wb, format=plsc.PackFormat.INTERLEAVED)
                        v0_b[b][s4] = g0
                        v1_b[b][s4] = g1
                    return c2

                lax.fori_loop(0, B // 64, unp_body, 0)

                @pl.when(i >= 1)
                def _():
                    wait_scat(o)

                @pl.when(jnp.logical_and(r == 0, i >= 1))
                def _():
                    pltpu.make_async_copy(
                        ones_buf, cnt_sp.at[dst_b[o]], c_sem[o]).wait()

                issue_idx(i + 1, o)
                pltpu.async_copy(v0_b[b], a0_sp.at[dst_b[b]], s_sem[b],
                                 add=True)
                pltpu.async_copy(v1_b[b], a1_sp.at[dst_b[b]], s_sem[b],
                                 add=True)

                @pl.when(r == 0)
                def _():
                    pltpu.async_copy(ones_buf, cnt_sp.at[dst_b[b]], c_sem[b],
                                     add=True)

            return c

        lax.fori_loop(0, NCHUNK // 2, chunk_pair, 0)
        wait_scat(1)
        wait_idx(0)

        @pl.when(r == 0)
        def _():
            pltpu.make_async_copy(ones_buf, cnt_sp.at[dst_b[1]],
                                  c_sem[1]).wait()

        plsc.subcore_barrier()

        @pl.when(r == 0)
        def _():
            pltpu.sync_copy(cnt_sp.at[pl.ds(n0, NC)], v0_c.at[pl.ds(0, NC)])

            def inv_body(j, c2):
                s4 = pl.ds(j * 16, 16)
                v0_c[s4] = 1.0 / jnp.maximum(v0_c[s4], 1.0)
                return c2

            lax.fori_loop(0, NC // 16, inv_body, 0)
            pltpu.sync_copy(v0_c.at[pl.ds(0, NC)], cnt_sp.at[pl.ds(n0, NC)])

        pltpu.sync_copy(a0_sp.at[pl.ds(n0, NC)], v0_a.at[pl.ds(0, NC)])
        pltpu.sync_copy(a1_sp.at[pl.ds(n0, NC)], v1_a.at[pl.ds(0, NC)])
        pltpu.sync_copy(cnt_sp.at[pl.ds(n0, NC)], v0_c.at[pl.ds(0, NC)])

        def norm_body(j, c2):
            s4 = pl.ds(j * 16, 16)
            iv = v0_c[s4]
            o0 = v0_a[s4] * iv
            o1 = v1_a[s4] * iv
            v0_a[s4] = o0
            v1_a[s4] = o1
            pk = plsc.pack(o0, o1, format=plsc.PackFormat.INTERLEAVED)
            vp_a[s4] = plsc.bitcast(pk, jnp.float32)
            return c2

        lax.fori_loop(0, NC // 16, norm_body, 0)
        ooff = (cid * N_ROUNDS + r) * NPAD + n0
        pltpu.sync_copy(v0_a.at[pl.ds(0, NC)], out0.at[pl.ds(ooff, NC)])
        pltpu.sync_copy(v1_a.at[pl.ds(0, NC)], out1.at[pl.ds(ooff, NC)])
        pltpu.sync_copy(vp_a.at[pl.ds(0, NC)], xp_sp.at[pl.ds(n0, NC)])
        plsc.subcore_barrier()
        return carry

    lax.fori_loop(0, N_ROUNDS, round_body, 0)


_mesh = plsc.VectorSubcoreMesh(core_axis_name="c", subcore_axis_name="s")

_dde_call = pl.kernel(
    _dde_body,
    out_type=(
        jax.ShapeDtypeStruct((2 * N_ROUNDS * NPAD,), jnp.float32),
        jax.ShapeDtypeStruct((2 * N_ROUNDS * NPAD,), jnp.float32),
    ),
    mesh=_mesh,
    compiler_params=pltpu.CompilerParams(needs_layout_passes=False),
    scratch_types=[
        pltpu.VMEM_SHARED((NPAD,), jnp.float32),
        pltpu.VMEM_SHARED((NPAD,), jnp.float32),
        pltpu.VMEM_SHARED((NPAD,), jnp.float32),
        pltpu.VMEM_SHARED((NPAD,), jnp.float32),
        pltpu.VMEM((B,), jnp.int32),
        pltpu.VMEM((B,), jnp.int32),
        pltpu.VMEM((B,), jnp.int32),
        pltpu.VMEM((B,), jnp.int32),
        pltpu.VMEM((B,), jnp.float32),
        pltpu.VMEM((B,), jnp.float32),
        pltpu.VMEM((B,), jnp.float32),
        pltpu.VMEM((B,), jnp.float32),
        pltpu.VMEM((B,), jnp.float32),
        pltpu.VMEM((B,), jnp.float32),
        pltpu.VMEM((B,), jnp.float32),
        pltpu.SemaphoreType.DMA,
        pltpu.SemaphoreType.DMA,
        pltpu.SemaphoreType.DMA,
        pltpu.SemaphoreType.DMA,
        pltpu.SemaphoreType.DMA,
        pltpu.SemaphoreType.DMA,
        pltpu.SemaphoreType.DMA,
        pltpu.SemaphoreType.DMA,
    ],
)


def kernel(topic_one_hot, edge_index, reverse_edge_index):
    bits0 = lax.bitcast_convert_type(
        topic_one_hot[:, 0].astype(jnp.bfloat16), jnp.uint16
    ).astype(jnp.uint32)
    bits1 = lax.bitcast_convert_type(
        topic_one_hot[:, 1].astype(jnp.bfloat16), jnp.uint16
    ).astype(jnp.uint32)
    packed = lax.bitcast_convert_type(
        jnp.left_shift(bits1, 16) | bits0, jnp.float32)
    xp = jnp.zeros((NPAD,), jnp.float32).at[:N_NODES].set(packed)
    srcs = jnp.concatenate([edge_index[0], reverse_edge_index[0]])
    dsts = jnp.concatenate([edge_index[1], reverse_edge_index[1]])
    ones = jnp.ones((B,), jnp.float32)
    zeros = jnp.zeros((NPAD,), jnp.float32)
    out0, out1 = _dde_call(xp, srcs, dsts, ones, zeros)
    out0 = out0.reshape(2, N_ROUNDS, NPAD)
    out1 = out1.reshape(2, N_ROUNDS, NPAD)
    res = []
    for c in range(2):
        for r in range(N_ROUNDS):
            res.append(jnp.stack([out0[c, r, :N_NODES], out1[c, r, :N_NODES]],
                                 axis=-1))
    return tuple(res)


# speedup vs baseline: 134.8989x; 1.0067x over previous
import jax
import jax.numpy as jnp
from jax import lax
from jax.experimental import pallas as pl
from jax.experimental.pallas import tpu as pltpu
from jax.experimental.pallas import tpu_sc as plsc

N_NODES = 100000
N_EDGES = 6400000
N_ROUNDS = 8
NTILES = 16
NPAD = 102400
NC = NPAD // NTILES
E_TILE = N_EDGES // NTILES
B = 8000
NCHUNK = E_TILE // B


def _dde_body(xp_hbm, srcs, dsts, ones, zeros, out0, out1,
              xp_sp, a0_sp, a1_sp, cnt_sp,
              src_a, src_c, dst_a, dst_c, vp_a, vp_c,
              v0_a, v0_c, v1_a, v1_c, ones_buf,
              i_sem_a, i_sem_c, g_sem_a, g_sem_c,
              s_sem_a, s_sem_c, c_sem_a, c_sem_c):
    cid = lax.axis_index("c")
    tid = lax.axis_index("s")
    n0 = tid * NC
    e_base = cid * N_EDGES + tid * E_TILE

    src_b = [src_a, src_c]
    dst_b = [dst_a, dst_c]
    vp_b = [vp_a, vp_c]
    v0_b = [v0_a, v0_c]
    v1_b = [v1_a, v1_c]
    i_sem = [i_sem_a, i_sem_c]
    g_sem = [g_sem_a, g_sem_c]
    s_sem = [s_sem_a, s_sem_c]
    c_sem = [c_sem_a, c_sem_c]

    def issue_idx(i, s):
        ic = jnp.minimum(i, NCHUNK - 1)
        e0 = e_base + ic * B
        pltpu.async_copy(srcs.at[pl.ds(e0, B)], src_b[s], i_sem[s])
        pltpu.async_copy(dsts.at[pl.ds(e0, B)], dst_b[s], i_sem[s])

    def wait_idx(s):
        pltpu.make_async_copy(srcs.at[pl.ds(0, B)], src_b[s], i_sem[s]).wait()
        pltpu.make_async_copy(dsts.at[pl.ds(0, B)], dst_b[s], i_sem[s]).wait()

    def wait_scat(s):
        pltpu.make_async_copy(v0_b[s], a0_sp.at[dst_b[s]], s_sem[s]).wait()
        pltpu.make_async_copy(v1_b[s], a1_sp.at[dst_b[s]], s_sem[s]).wait()

    pltpu.sync_copy(xp_hbm.at[pl.ds(n0, NC)], vp_a.at[pl.ds(0, NC)])
    pltpu.sync_copy(vp_a.at[pl.ds(0, NC)], xp_sp.at[pl.ds(n0, NC)])
    pltpu.sync_copy(zeros.at[pl.ds(n0, NC)], v0_a.at[pl.ds(0, NC)])
    pltpu.sync_copy(v0_a.at[pl.ds(0, NC)], cnt_sp.at[pl.ds(n0, NC)])
    pltpu.sync_copy(ones, ones_buf)
    plsc.subcore_barrier()

    def round_body(r, carry):
        pltpu.sync_copy(zeros.at[pl.ds(n0, NC)], v0_a.at[pl.ds(0, NC)])
        pltpu.sync_copy(v0_a.at[pl.ds(0, NC)], a0_sp.at[pl.ds(n0, NC)])
        pltpu.sync_copy(v0_a.at[pl.ds(0, NC)], a1_sp.at[pl.ds(n0, NC)])
        plsc.subcore_barrier()
        issue_idx(0, 0)

        def chunk_pair(g, c):
            for b in (0, 1):
                i = 2 * g + b
                o = 1 - b
                wait_idx(b)
                H = B // 2
                sh0 = src_b[b].at[pl.ds(0, H)]
                sh1 = src_b[b].at[pl.ds(H, H)]
                vh0 = vp_b[b].at[pl.ds(0, H)]
                vh1 = vp_b[b].at[pl.ds(H, H)]
                pltpu.async_copy(xp_sp.at[sh0], vh0, g_sem[b])
                pltpu.async_copy(xp_sp.at[sh1], vh1, g_sem[b])
                pltpu.make_async_copy(xp_sp.at[sh0], vh0, g_sem[b]).wait()
                pltpu.make_async_copy(xp_sp.at[sh1], vh1, g_sem[b]).wait()

                def unp_body(k, c2):
                    for u in range(4):
                        s4 = pl.ds(k * 64 + u * 16, 16)
                        wb = plsc.bitcast(vp_b[b][s4], jnp.bfloat16)
                        g0, g1 = plsc.unpack(
                            wb, format=plsc.PackFormat.INTERLEAVED)
                        v0_b[b][s4] = g0
                        v1_b[b][s4] = g1
                    return c2

                lax.fori_loop(0, B // 64, unp_body, 0)

                @pl.when(i >= 1)
                def _():
                    wait_scat(o)

                @pl.when(jnp.logical_and(r == 0, i >= 1))
                def _():
                    pltpu.make_async_copy(
                        ones_buf, cnt_sp.at[dst_b[o]], c_sem[o]).wait()

                issue_idx(i + 1, o)
                pltpu.async_copy(v0_b[b], a0_sp.at[dst_b[b]], s_sem[b],
                                 add=True)
                pltpu.async_copy(v1_b[b], a1_sp.at[dst_b[b]], s_sem[b],
                                 add=True)

                @pl.when(r == 0)
                def _():
                    pltpu.async_copy(ones_buf, cnt_sp.at[dst_b[b]], c_sem[b],
                                     add=True)

            return c

        lax.fori_loop(0, NCHUNK // 2, chunk_pair, 0)
        wait_scat(1)
        wait_idx(0)

        @pl.when(r == 0)
        def _():
            pltpu.make_async_copy(ones_buf, cnt_sp.at[dst_b[1]],
                                  c_sem[1]).wait()

        plsc.subcore_barrier()

        @pl.when(r == 0)
        def _():
            pltpu.sync_copy(cnt_sp.at[pl.ds(n0, NC)], v0_c.at[pl.ds(0, NC)])

            def inv_body(j, c2):
                s4 = pl.ds(j * 16, 16)
                v0_c[s4] = 1.0 / jnp.maximum(v0_c[s4], 1.0)
                return c2

            lax.fori_loop(0, NC // 16, inv_body, 0)
            pltpu.sync_copy(v0_c.at[pl.ds(0, NC)], cnt_sp.at[pl.ds(n0, NC)])

        pltpu.sync_copy(a0_sp.at[pl.ds(n0, NC)], v0_a.at[pl.ds(0, NC)])
        pltpu.sync_copy(a1_sp.at[pl.ds(n0, NC)], v1_a.at[pl.ds(0, NC)])
        pltpu.sync_copy(cnt_sp.at[pl.ds(n0, NC)], v0_c.at[pl.ds(0, NC)])

        def norm_body(j, c2):
            s4 = pl.ds(j * 16, 16)
            iv = v0_c[s4]
            o0 = v0_a[s4] * iv
            o1 = v1_a[s4] * iv
            v0_a[s4] = o0
            v1_a[s4] = o1
            pk = plsc.pack(o0, o1, format=plsc.PackFormat.INTERLEAVED)
            vp_a[s4] = plsc.bitcast(pk, jnp.float32)
            return c2

        lax.fori_loop(0, NC // 16, norm_body, 0)
        ooff = (cid * N_ROUNDS + r) * NPAD + n0
        pltpu.sync_copy(v0_a.at[pl.ds(0, NC)], out0.at[pl.ds(ooff, NC)])
        pltpu.sync_copy(v1_a.at[pl.ds(0, NC)], out1.at[pl.ds(ooff, NC)])
        pltpu.sync_copy(vp_a.at[pl.ds(0, NC)], xp_sp.at[pl.ds(n0, NC)])
        plsc.subcore_barrier()
        return carry

    lax.fori_loop(0, N_ROUNDS, round_body, 0)


_mesh = plsc.VectorSubcoreMesh(core_axis_name="c", subcore_axis_name="s")

_dde_call = pl.kernel(
    _dde_body,
    out_type=(
        jax.ShapeDtypeStruct((2 * N_ROUNDS * NPAD,), jnp.float32),
        jax.ShapeDtypeStruct((2 * N_ROUNDS * NPAD,), jnp.float32),
    ),
    mesh=_mesh,
    compiler_params=pltpu.CompilerParams(needs_layout_passes=False),
    scratch_types=[
        pltpu.VMEM_SHARED((NPAD,), jnp.float32),
        pltpu.VMEM_SHARED((NPAD,), jnp.float32),
        pltpu.VMEM_SHARED((NPAD,), jnp.float32),
        pltpu.VMEM_SHARED((NPAD,), jnp.float32),
        pltpu.VMEM((B,), jnp.int32),
        pltpu.VMEM((B,), jnp.int32),
        pltpu.VMEM((B,), jnp.int32),
        pltpu.VMEM((B,), jnp.int32),
        pltpu.VMEM((B,), jnp.float32),
        pltpu.VMEM((B,), jnp.float32),
        pltpu.VMEM((B,), jnp.float32),
        pltpu.VMEM((B,), jnp.float32),
        pltpu.VMEM((B,), jnp.float32),
        pltpu.VMEM((B,), jnp.float32),
        pltpu.VMEM((B,), jnp.float32),
        pltpu.SemaphoreType.DMA,
        pltpu.SemaphoreType.DMA,
        pltpu.SemaphoreType.DMA,
        pltpu.SemaphoreType.DMA,
        pltpu.SemaphoreType.DMA,
        pltpu.SemaphoreType.DMA,
        pltpu.SemaphoreType.DMA,
        pltpu.SemaphoreType.DMA,
    ],
)


def kernel(topic_one_hot, edge_index, reverse_edge_index):
    bits0 = lax.bitcast_convert_type(
        topic_one_hot[:, 0].astype(jnp.bfloat16), jnp.uint16
    ).astype(jnp.uint32)
    bits1 = lax.bitcast_convert_type(
        topic_one_hot[:, 1].astype(jnp.bfloat16), jnp.uint16
    ).astype(jnp.uint32)
    packed = lax.bitcast_convert_type(
        jnp.left_shift(bits1, 16) | bits0, jnp.float32)
    xp = jnp.zeros((NPAD,), jnp.float32).at[:N_NODES].set(packed)
    srcs = jnp.concatenate([edge_index[0], reverse_edge_index[0]])
    dsts = jnp.concatenate([edge_index[1], reverse_edge_index[1]])
    ones = jnp.ones((B,), jnp.float32)
    zeros = jnp.zeros((NPAD,), jnp.float32)
    out0, out1 = _dde_call(xp, srcs, dsts, ones, zeros)
    out0 = out0.reshape(2, N_ROUNDS, NPAD)
    out1 = out1.reshape(2, N_ROUNDS, NPAD)
    res = []
    for c in range(2):
        for r in range(N_ROUNDS):
            res.append(jnp.stack([out0[c, r, :N_NODES], out1[c, r, :N_NODES]],
                                 axis=-1))
    return tuple(res)


# speedup vs baseline: 143.6578x; 1.0649x over previous
import jax
import jax.numpy as jnp
from jax import lax
from jax.experimental import pallas as pl
from jax.experimental.pallas import tpu as pltpu
from jax.experimental.pallas import tpu_sc as plsc

N_NODES = 100000
N_EDGES = 6400000
N_ROUNDS = 8
NTILES = 16
NPAD = 102400
NC = NPAD // NTILES
E_TILE = N_EDGES // NTILES
B = 10000
NCHUNK = E_TILE // B


def _dde_body(xp_hbm, srcs, dsts, ones, zeros, out0, out1,
              xp_sp, a0_sp, a1_sp, cnt_sp,
              src_a, src_c, dst_a, dst_c,
              v0_a, v0_c, v1_a, v1_c, ones_buf, pk_buf,
              i_sem_a, i_sem_c, g_sem_a, g_sem_c,
              s_sem_a, s_sem_c, c_sem_a, c_sem_c):
    cid = lax.axis_index("c")
    tid = lax.axis_index("s")
    n0 = tid * NC
    e_base = cid * N_EDGES + tid * E_TILE

    src_b = [src_a, src_c]
    dst_b = [dst_a, dst_c]
    v0_b = [v0_a, v0_c]
    v1_b = [v1_a, v1_c]
    i_sem = [i_sem_a, i_sem_c]
    g_sem = [g_sem_a, g_sem_c]
    s_sem = [s_sem_a, s_sem_c]
    c_sem = [c_sem_a, c_sem_c]

    def issue_idx(i, s):
        ic = jnp.minimum(i, NCHUNK - 1)
        e0 = e_base + ic * B
        pltpu.async_copy(srcs.at[pl.ds(e0, B)], src_b[s], i_sem[s])
        pltpu.async_copy(dsts.at[pl.ds(e0, B)], dst_b[s], i_sem[s])

    def wait_idx(s):
        pltpu.make_async_copy(srcs.at[pl.ds(0, B)], src_b[s], i_sem[s]).wait()
        pltpu.make_async_copy(dsts.at[pl.ds(0, B)], dst_b[s], i_sem[s]).wait()

    def wait_scat(s):
        pltpu.make_async_copy(v0_b[s], a0_sp.at[dst_b[s]], s_sem[s]).wait()
        pltpu.make_async_copy(v1_b[s], a1_sp.at[dst_b[s]], s_sem[s]).wait()

    pltpu.sync_copy(xp_hbm.at[pl.ds(n0, NC)], pk_buf.at[pl.ds(0, NC)])
    pltpu.sync_copy(pk_buf.at[pl.ds(0, NC)], xp_sp.at[pl.ds(n0, NC)])
    pltpu.sync_copy(zeros.at[pl.ds(n0, NC)], v0_a.at[pl.ds(0, NC)])
    pltpu.sync_copy(v0_a.at[pl.ds(0, NC)], cnt_sp.at[pl.ds(n0, NC)])
    pltpu.sync_copy(ones, ones_buf)
    plsc.subcore_barrier()

    def round_body(r, carry):
        pltpu.sync_copy(zeros.at[pl.ds(n0, NC)], v0_a.at[pl.ds(0, NC)])
        pltpu.sync_copy(v0_a.at[pl.ds(0, NC)], a0_sp.at[pl.ds(n0, NC)])
        pltpu.sync_copy(v0_a.at[pl.ds(0, NC)], a1_sp.at[pl.ds(n0, NC)])
        plsc.subcore_barrier()
        issue_idx(0, 0)

        def chunk_pair(g, c):
            for b in (0, 1):
                i = 2 * g + b
                o = 1 - b
                wait_idx(b)
                H = B // 2
                sh0 = src_b[b].at[pl.ds(0, H)]
                sh1 = src_b[b].at[pl.ds(H, H)]
                vh0 = v0_b[b].at[pl.ds(0, H)]
                vh1 = v0_b[b].at[pl.ds(H, H)]
                pltpu.async_copy(xp_sp.at[sh0], vh0, g_sem[b])
                pltpu.async_copy(xp_sp.at[sh1], vh1, g_sem[b])
                pltpu.make_async_copy(xp_sp.at[sh0], vh0, g_sem[b]).wait()
                pltpu.make_async_copy(xp_sp.at[sh1], vh1, g_sem[b]).wait()

                def unp_body(k, c2):
                    for u in range(4):
                        s4 = pl.ds(k * 64 + u * 16, 16)
                        wb = plsc.bitcast(v0_b[b][s4], jnp.bfloat16)
                        g0, g1 = plsc.unpack(
                            wb, format=plsc.PackFormat.INTERLEAVED)
                        v0_b[b][s4] = g0
                        v1_b[b][s4] = g1
                    return c2

                lax.fori_loop(0, B // 64, unp_body, 0)

                @pl.when(i >= 1)
                def _():
                    wait_scat(o)

                @pl.when(jnp.logical_and(r == 0, i >= 1))
                def _():
                    pltpu.make_async_copy(
                        ones_buf, cnt_sp.at[dst_b[o]], c_sem[o]).wait()

                issue_idx(i + 1, o)
                pltpu.async_copy(v0_b[b], a0_sp.at[dst_b[b]], s_sem[b],
                                 add=True)
                pltpu.async_copy(v1_b[b], a1_sp.at[dst_b[b]], s_sem[b],
                                 add=True)

                @pl.when(r == 0)
                def _():
                    pltpu.async_copy(ones_buf, cnt_sp.at[dst_b[b]], c_sem[b],
                                     add=True)

            return c

        lax.fori_loop(0, NCHUNK // 2, chunk_pair, 0)
        wait_scat(1)
        wait_idx(0)

        @pl.when(r == 0)
        def _():
            pltpu.make_async_copy(ones_buf, cnt_sp.at[dst_b[1]],
                                  c_sem[1]).wait()

        plsc.subcore_barrier()

        @pl.when(r == 0)
        def _():
            pltpu.sync_copy(cnt_sp.at[pl.ds(n0, NC)], v0_c.at[pl.ds(0, NC)])

            def inv_body(j, c2):
                s4 = pl.ds(j * 16, 16)
                v0_c[s4] = 1.0 / jnp.maximum(v0_c[s4], 1.0)
                return c2

            lax.fori_loop(0, NC // 16, inv_body, 0)
            pltpu.sync_copy(v0_c.at[pl.ds(0, NC)], cnt_sp.at[pl.ds(n0, NC)])

        pltpu.sync_copy(a0_sp.at[pl.ds(n0, NC)], v0_a.at[pl.ds(0, NC)])
        pltpu.sync_copy(a1_sp.at[pl.ds(n0, NC)], v1_a.at[pl.ds(0, NC)])
        pltpu.sync_copy(cnt_sp.at[pl.ds(n0, NC)], v0_c.at[pl.ds(0, NC)])

        def norm_body(j, c2):
            s4 = pl.ds(j * 16, 16)
            iv = v0_c[s4]
            o0 = v0_a[s4] * iv
            o1 = v1_a[s4] * iv
            v0_a[s4] = o0
            v1_a[s4] = o1
            pk = plsc.pack(o0, o1, format=plsc.PackFormat.INTERLEAVED)
            pk_buf[s4] = plsc.bitcast(pk, jnp.float32)
            return c2

        lax.fori_loop(0, NC // 16, norm_body, 0)
        ooff = (cid * N_ROUNDS + r) * NPAD + n0
        pltpu.sync_copy(v0_a.at[pl.ds(0, NC)], out0.at[pl.ds(ooff, NC)])
        pltpu.sync_copy(v1_a.at[pl.ds(0, NC)], out1.at[pl.ds(ooff, NC)])
        pltpu.sync_copy(pk_buf.at[pl.ds(0, NC)], xp_sp.at[pl.ds(n0, NC)])
        plsc.subcore_barrier()
        return carry

    lax.fori_loop(0, N_ROUNDS, round_body, 0)


_mesh = plsc.VectorSubcoreMesh(core_axis_name="c", subcore_axis_name="s")

_dde_call = pl.kernel(
    _dde_body,
    out_type=(
        jax.ShapeDtypeStruct((2 * N_ROUNDS * NPAD,), jnp.float32),
        jax.ShapeDtypeStruct((2 * N_ROUNDS * NPAD,), jnp.float32),
    ),
    mesh=_mesh,
    compiler_params=pltpu.CompilerParams(needs_layout_passes=False),
    scratch_types=[
        pltpu.VMEM_SHARED((NPAD,), jnp.float32),
        pltpu.VMEM_SHARED((NPAD,), jnp.float32),
        pltpu.VMEM_SHARED((NPAD,), jnp.float32),
        pltpu.VMEM_SHARED((NPAD,), jnp.float32),
        pltpu.VMEM((B,), jnp.int32),
        pltpu.VMEM((B,), jnp.int32),
        pltpu.VMEM((B,), jnp.int32),
        pltpu.VMEM((B,), jnp.int32),
        pltpu.VMEM((B,), jnp.float32),
        pltpu.VMEM((B,), jnp.float32),
        pltpu.VMEM((B,), jnp.float32),
        pltpu.VMEM((B,), jnp.float32),
        pltpu.VMEM((B,), jnp.float32),
        pltpu.VMEM((B,), jnp.float32),
        pltpu.SemaphoreType.DMA,
        pltpu.SemaphoreType.DMA,
        pltpu.SemaphoreType.DMA,
        pltpu.SemaphoreType.DMA,
        pltpu.SemaphoreType.DMA,
        pltpu.SemaphoreType.DMA,
        pltpu.SemaphoreType.DMA,
        pltpu.SemaphoreType.DMA,
    ],
)


def kernel(topic_one_hot, edge_index, reverse_edge_index):
    bits0 = lax.bitcast_convert_type(
        topic_one_hot[:, 0].astype(jnp.bfloat16), jnp.uint16
    ).astype(jnp.uint32)
    bits1 = lax.bitcast_convert_type(
        topic_one_hot[:, 1].astype(jnp.bfloat16), jnp.uint16
    ).astype(jnp.uint32)
    packed = lax.bitcast_convert_type(
        jnp.left_shift(bits1, 16) | bits0, jnp.float32)
    xp = jnp.zeros((NPAD,), jnp.float32).at[:N_NODES].set(packed)
    srcs = jnp.concatenate([edge_index[0], reverse_edge_index[0]])
    dsts = jnp.concatenate([edge_index[1], reverse_edge_index[1]])
    ones = jnp.ones((B,), jnp.float32)
    zeros = jnp.zeros((NPAD,), jnp.float32)
    out0, out1 = _dde_call(xp, srcs, dsts, ones, zeros)
    out0 = out0.reshape(2, N_ROUNDS, NPAD)
    out1 = out1.reshape(2, N_ROUNDS, NPAD)
    res = []
    for c in range(2):
        for r in range(N_ROUNDS):
            res.append(jnp.stack([out0[c, r, :N_NODES], out1[c, r, :N_NODES]],
                                 axis=-1))
    return tuple(res)
